# baseline (speedup 1.0000x reference)
import functools
import jax
import jax.numpy as jnp
from jax import lax
from jax.experimental import pallas as pl
from jax.experimental.pallas import tpu as pltpu
from jax.experimental.pallas import tpu_sc as plsc

_N = 10000
_D = 128
_K = 4
_NE = 100000
_NP = 10240
_EP = 170000
_NB = 128
_GN = _NP // _NB
_CH = 6800
_NCH = _EP // _CH
_F1 = 256

_CP_SC = pltpu.CompilerParams(needs_layout_passes=False)


@functools.cache
def _mesh():
    return plsc.VectorSubcoreMesh(core_axis_name="c", subcore_axis_name="s")
_HI = jax.lax.Precision.HIGHEST


def _wid():
    return lax.axis_index("s") * 2 + lax.axis_index("c")


def _dot(a, b, dims):
    return lax.dot_general(a, b, (dims, ((), ())),
                           preferred_element_type=jnp.float32, precision=_HI)


_GPT = 1280
_GCH = 128


def _sc_gather(tbl_hbm, idx_hbm, out_hbm, idx_v, rows_v, sem):
    w = _wid()
    base = w * _GPT
    pltpu.sync_copy(idx_hbm.at[pl.ds(base, _GPT)], idx_v)

    def body(c, carry):
        pltpu.async_copy(tbl_hbm.at[idx_v.at[pl.ds(c * _GCH, _GCH)]],
                         rows_v, sem).wait()
        pltpu.sync_copy(rows_v, out_hbm.at[pl.ds(base + c * _GCH, _GCH)])
        return carry

    lax.fori_loop(0, _GPT // _GCH, body, 0)


@jax.jit
def _gather_embeddings(tables, entity_ids):
    idx = (entity_ids.astype(jnp.int32)[None, :]
           + (jnp.arange(_K, dtype=jnp.int32) * _NE)[:, None]).reshape(-1)
    idx = jnp.pad(idx, (0, 32 * _GPT - _K * _N))
    k = functools.partial(
        pl.kernel, mesh=_mesh(), compiler_params=_CP_SC,
        out_type=jax.ShapeDtypeStruct((32 * _GPT, _D), jnp.float32),
        scratch_types=[
            pltpu.VMEM((_GPT,), jnp.int32),
            pltpu.VMEM((_GCH, _D), jnp.float32),
            pltpu.SemaphoreType.DMA,
        ],
    )(_sc_gather)
    rows = k(tables.reshape(_K * _NE, _D), idx)
    emb = rows[:_K * _N].reshape(_K, _N, _D)
    return jnp.pad(emb, ((0, 0), (0, _NP - _N), (0, 0)))


def _edge_loop(src_hbm, dst_hbm, si_v, di_v, es_v, ed_v, xlT_v, acc_v,
               den_v):
    fcs = [jnp.full((16,), f, jnp.int32) for f in range(4)]

    def chunk(c, carry):
        pltpu.sync_copy(src_hbm.at[pl.ds(c * _CH, _CH)], si_v)
        pltpu.sync_copy(dst_hbm.at[pl.ds(c * _CH, _CH)], di_v)

        def vec(jb, carry2):
            for u in range(5):
                off = (jb * 5 + u) * 16
                sv = si_v[pl.ds(off, 16)]
                dv = di_v[pl.ds(off, 16)]
                es_g = plsc.load_gather(es_v, [sv])
                ed_g = plsc.load_gather(ed_v, [dv])
                e = es_g + ed_g
                wv = jnp.exp(jnp.maximum(e, 0.2 * e))
                plsc.addupdate_scatter(den_v, [dv], wv)
                for f in range(4):
                    xv = plsc.load_gather(xlT_v, [fcs[f], sv])
                    plsc.addupdate_scatter(acc_v, [fcs[f], dv], wv * xv)
            return carry2

        lax.fori_loop(0, (_CH // 16) // 5, vec, 0)
        return carry

    lax.fori_loop(0, _NCH, chunk, 0)


def _edge_unit(esd_hbm, xlT_hbm, src_hbm, dst_hbm, zeros_hbm, num_hbm,
               den_hbm, xlT_v, acc_v, es_v, ed_v, den_v, si_v, di_v,
               k, f0, h_es, h_ed, h_den, write_den):
    pltpu.sync_copy(esd_hbm.at[k, h_es], es_v)
    pltpu.sync_copy(esd_hbm.at[k, h_ed], ed_v)
    pltpu.sync_copy(xlT_hbm.at[k, pl.ds(f0, 4)], xlT_v)
    for r in range(4):
        pltpu.sync_copy(zeros_hbm, acc_v.at[r])
    pltpu.sync_copy(zeros_hbm, den_v)
    _edge_loop(src_hbm, dst_hbm, si_v, di_v, es_v, ed_v, xlT_v, acc_v,
               den_v)
    pltpu.sync_copy(acc_v, num_hbm.at[k, pl.ds(f0, 4)])

    @pl.when(write_den)
    def _():
        pltpu.sync_copy(den_v, den_hbm.at[k, h_den])


def _sc_edge1(xlT_hbm, esd_hbm, src_hbm, dst_hbm, zeros_hbm,
              num_hbm, den_hbm, xlT_v, acc_v, es_v, ed_v, den_v,
              si_v, di_v):
    w = _wid()
    for p in range(2):
        f0 = p * 128 + w * 4
        h = f0 // 32
        for k in range(_K):
            _edge_unit(esd_hbm, xlT_hbm, src_hbm, dst_hbm, zeros_hbm,
                       num_hbm, den_hbm, xlT_v, acc_v, es_v, ed_v, den_v,
                       si_v, di_v, k, f0, h, h + 8, h, f0 % 32 == 0)


def _sc_edge2(xlT_hbm, esd_hbm, src_hbm, dst_hbm, zeros_hbm,
              num_hbm, den_hbm, xlT_v, acc_v, es_v, ed_v, den_v,
              si_v, di_v):
    w = _wid()
    f0 = w * 4
    for k in range(_K):
        _edge_unit(esd_hbm, xlT_hbm, src_hbm, dst_hbm, zeros_hbm,
                   num_hbm, den_hbm, xlT_v, acc_v, es_v, ed_v, den_v,
                   si_v, di_v, k, f0, 0, 1, 0, f0 == 0)


def _edge_call(body, f_total, src, dst, xlT, esd, nheads):
    k = functools.partial(
        pl.kernel, mesh=_mesh(), compiler_params=_CP_SC,
        out_type=[
            jax.ShapeDtypeStruct((_K, f_total, _NP), jnp.float32),
            jax.ShapeDtypeStruct((_K, nheads, _NP), jnp.float32),
        ],
        scratch_types=[
            pltpu.VMEM((4, _NP), jnp.float32),
            pltpu.VMEM((4, _NP), jnp.float32),
            pltpu.VMEM((_NP,), jnp.float32),
            pltpu.VMEM((_NP,), jnp.float32),
            pltpu.VMEM((_NP,), jnp.float32),
            pltpu.VMEM((_CH,), jnp.int32),
            pltpu.VMEM((_CH,), jnp.int32),
        ],
    )(body)
    zeros = jnp.zeros((_NP,), jnp.float32)
    return k(xlT, esd, src, dst, zeros)


def _ln_t(x, g, b):
    m = jnp.mean(x, axis=0, keepdims=True)
    v = jnp.mean((x - m) ** 2, axis=0, keepdims=True)
    return (x - m) / jnp.sqrt(v + 1e-5) * g + b


def _elu(x):
    return jnp.where(x > 0, x, jnp.exp(jnp.minimum(x, 0.0)) - 1.0)


def _k2_body(emb_ref, w1_ref, a1_ref, xlT_ref, esd_ref):
    e = emb_ref[0]
    xlT = _dot(w1_ref[0], e, ((0,), (1,)))
    xlT_ref[0] = xlT
    esd_ref[0] = _dot(a1_ref[0], xlT, ((0,), (0,)))


@jax.jit
def _tc_k2(emb, W1, A1):
    return pl.pallas_call(
        _k2_body,
        grid=(_K, _GN),
        in_specs=[
            pl.BlockSpec((1, _NB, _D), lambda k, n: (k, n, 0)),
            pl.BlockSpec((1, _D, _F1), lambda k, n: (k, 0, 0)),
            pl.BlockSpec((1, _F1, 16), lambda k, n: (k, 0, 0)),
        ],
        out_specs=[
            pl.BlockSpec((1, _F1, _NB), lambda k, n: (k, 0, n)),
            pl.BlockSpec((1, 16, _NB), lambda k, n: (k, 0, n)),
        ],
        out_shape=[
            jax.ShapeDtypeStruct((_K, _F1, _NP), jnp.float32),
            jax.ShapeDtypeStruct((_K, 16, _NP), jnp.float32),
        ],
    )(emb, W1, A1)


def _k4_body(num_ref, den_ref, e8_ref, b1_ref, g1_ref, bb1_ref, w2_ref,
             a2_ref, xl2_ref, esd2_ref):
    dene = _dot(e8_ref[...], den_ref[0], ((1,), (0,)))
    o = num_ref[0] / (dene + 1e-16) + b1_ref[0]
    h = _elu(_ln_t(o, g1_ref[0], bb1_ref[0]))
    xl2 = _dot(w2_ref[0], h, ((0,), (0,)))
    xl2_ref[0] = xl2
    esd2_ref[0] = _dot(a2_ref[0], xl2, ((0,), (0,)))


@jax.jit
def _tc_k4(num1, den1, E8, b1c, ln1gc, ln1bc, W2, A2):
    return pl.pallas_call(
        _k4_body,
        grid=(_K, _GN),
        in_specs=[
            pl.BlockSpec((1, _F1, _NB), lambda k, n: (k, 0, n)),
            pl.BlockSpec((1, 8, _NB), lambda k, n: (k, 0, n)),
            pl.BlockSpec((_F1, 8), lambda k, n: (0, 0)),
            pl.BlockSpec((1, _F1, 1), lambda k, n: (k, 0, 0)),
            pl.BlockSpec((1, _F1, 1), lambda k, n: (k, 0, 0)),
            pl.BlockSpec((1, _F1, 1), lambda k, n: (k, 0, 0)),
            pl.BlockSpec((1, _F1, _D), lambda k, n: (k, 0, 0)),
            pl.BlockSpec((1, _D, 16), lambda k, n: (k, 0, 0)),
        ],
        out_specs=[
            pl.BlockSpec((1, _D, _NB), lambda k, n: (k, 0, n)),
            pl.BlockSpec((1, 16, _NB), lambda k, n: (k, 0, n)),
        ],
        out_shape=[
            jax.ShapeDtypeStruct((_K, _D, _NP), jnp.float32),
            jax.ShapeDtypeStruct((_K, 16, _NP), jnp.float32),
        ],
    )(num1, den1, E8, b1c, ln1gc, ln1bc, W2, A2)


def _k6_body(num_ref, den_ref, b2_ref, g2_ref, bb2_ref, pw1_ref, pb1_ref,
             pg_ref, pbb_ref, pw2_ref, pb2_ref, kg_ref, mw_ref, mb_ref,
             gx_ref, gc_ref, gb_ref, s8_ref, fw1_ref, fb1_ref, fg_ref,
             fbb_ref, fw2_ref, fb2_ref, i128_ref, out_ref):
    s8 = s8_ref[...]
    xs = []
    for k in range(_K):
        o2 = num_ref[k] / (den_ref[k] + 1e-16) + b2_ref[k]
        h2 = _elu(_ln_t(o2, g2_ref[k], bb2_ref[k]))
        t = _dot(pw1_ref[k], h2, ((0,), (0,))) + pb1_ref[k]
        t = jnp.maximum(_ln_t(t, pg_ref[k], pbb_ref[k]), 0.0)
        p = _dot(pw2_ref[k], t, ((0,), (0,))) + pb2_ref[k]
        xs.append(p + kg_ref[k])

    def mha(seq, layer):
        qs = [_dot(mw_ref[layer, 0], x, ((0,), (0,))) + mb_ref[layer, 0]
              for x in seq]
        ks = [_dot(mw_ref[layer, 1], x, ((0,), (0,))) + mb_ref[layer, 1]
              for x in seq]
        vs = [_dot(mw_ref[layer, 2], x, ((0,), (0,))) + mb_ref[layer, 2]
              for x in seq]
        outs = []
        for i in range(4):
            sc = [_dot(s8, qs[i] * ks[j], ((0,), (0,))) * 0.25
                  for j in range(4)]
            m = jnp.maximum(jnp.maximum(sc[0], sc[1]),
                            jnp.maximum(sc[2], sc[3]))
            ex = [jnp.exp(s - m) for s in sc]
            tot = ex[0] + ex[1] + ex[2] + ex[3]
            o = sum(_dot(s8, ex[j] / tot, ((1,), (0,))) * vs[j]
                    for j in range(4))
            outs.append(_dot(mw_ref[layer, 3], o, ((0,), (0,)))
                        + mb_ref[layer, 3])
        return outs

    intra = mha(xs, 0)
    cross = mha(intra, 1)
    fused = []
    for s in range(4):
        g = jax.nn.sigmoid(_dot(gx_ref[...], xs[s], ((0,), (0,)))
                           + _dot(gc_ref[...], cross[s], ((0,), (0,)))
                           + gb_ref[...])
        fused.append(g * cross[s] + (1.0 - g) * xs[s])
    fcat = jnp.concatenate(fused, axis=0)
    t = _dot(fw1_ref[...], fcat, ((0,), (0,))) + fb1_ref[...]
    t = jnp.maximum(_ln_t(t, fg_ref[...], fbb_ref[...]), 0.0)
    oT = _dot(fw2_ref[...], t, ((0,), (0,))) + fb2_ref[...]
    out_ref[...] = _dot(oT, i128_ref[...], ((0,), (0,)))


@jax.jit
def _tc_k6(num2, den2, b2c, ln2gc, ln2bc, pw1, pb1c, plngc, plnbc, pw2,
           pb2c, kgc, mha_w, mha_bc, gx, gc, gbc, S8, fw1, fb1c, flngc,
           flnbc, fw2, fb2c, I128):
    full = lambda *shape: pl.BlockSpec(shape, lambda n: (0,) * len(shape))
    return pl.pallas_call(
        _k6_body,
        grid=(_GN,),
        in_specs=[
            pl.BlockSpec((_K, _D, _NB), lambda n: (0, 0, n)),
            pl.BlockSpec((_K, 1, _NB), lambda n: (0, 0, n)),
            full(_K, _D, 1), full(_K, _D, 1), full(_K, _D, 1),
            full(_K, _D, 32), full(_K, 32, 1), full(_K, 32, 1),
            full(_K, 32, 1), full(_K, 32, _D), full(_K, _D, 1),
            full(_K, _D, 1),
            full(2, 4, _D, _D), full(2, 4, _D, 1),
            full(_D, _D), full(_D, _D), full(_D, 1),
            full(_D, 8),
            full(4 * _D, 32), full(32, 1), full(32, 1), full(32, 1),
            full(32, _D), full(_D, 1),
            full(_D, _D),
        ],
        out_specs=pl.BlockSpec((_NB, _D), lambda n: (n, 0)),
        out_shape=jax.ShapeDtypeStruct((_NP, _D), jnp.float32),
    )(num2, den2, b2c, ln2gc, ln2bc, pw1, pb1c, plngc, plnbc, pw2, pb2c,
      kgc, mha_w, mha_bc, gx, gc, gbc, S8, fw1, fb1c, flngc, flnbc, fw2,
      fb2c, I128)


def kernel(entity_ids, edge_index, tables, W1, b1, a1s, a1d, ln1g, ln1b,
           W2, b2, a2s, a2d, ln2g, ln2b, pw1, pb1, plng, plnb, pw2, pb2,
           kg_emb, mha_w, mha_b, gate_w, gate_b, fw1, fb1, flng, flnb,
           fw2, fb2):
    f32 = jnp.float32
    src = jnp.concatenate([edge_index[0].astype(jnp.int32),
                           jnp.arange(_N, dtype=jnp.int32)])
    dst = jnp.concatenate([edge_index[1].astype(jnp.int32),
                           jnp.arange(_N, dtype=jnp.int32)])

    eye8 = jnp.eye(8, dtype=f32)
    A1 = jnp.concatenate([
        (a1s[:, :, :, None] * eye8[:, None, :]).reshape(_K, _F1, 8),
        (a1d[:, :, :, None] * eye8[:, None, :]).reshape(_K, _F1, 8),
    ], axis=2)
    A2 = jnp.zeros((_K, _D, 16), f32)
    A2 = A2.at[:, :, 0].set(a2s[:, 0, :]).at[:, :, 1].set(a2d[:, 0, :])
    E8 = jnp.kron(jnp.eye(8, dtype=f32), jnp.ones((32, 1), f32))
    S8 = jnp.kron(jnp.eye(8, dtype=f32), jnp.ones((16, 1), f32))
    I128 = jnp.eye(_D, dtype=f32)

    col = lambda a: a[..., None]

    emb = _gather_embeddings(tables, entity_ids)

    xl1T, esd1 = _tc_k2(emb, W1, A1)
    num1, den1 = _edge_call(_sc_edge1, _F1, src, dst, xl1T, esd1, 8)
    xl2T, esd2 = _tc_k4(num1, den1, E8, col(b1), col(ln1g), col(ln1b),
                        W2, A2)
    num2, den2 = _edge_call(_sc_edge2, _D, src, dst, xl2T, esd2, 1)

    o = _tc_k6(num2, den2, col(b2), col(ln2g), col(ln2b),
               pw1, col(pb1), col(plng), col(plnb), pw2, col(pb2),
               col(kg_emb), mha_w, col(mha_b), gate_w[:_D], gate_w[_D:],
               col(gate_b), S8, fw1, col(fb1), col(flng), col(flnb),
               fw2, col(fb2), I128)
    return o[:_N]


# speedup vs baseline: 1.2887x; 1.2887x over previous
import functools
import jax
import jax.numpy as jnp
from jax import lax
from jax.experimental import pallas as pl
from jax.experimental.pallas import tpu as pltpu
from jax.experimental.pallas import tpu_sc as plsc

_N = 10000
_D = 128
_K = 4
_NE = 100000
_NP = 10240
_EP = 170000
_NB = 128
_GN = _NP // _NB
_CH = 6800
_NCH = _EP // _CH
_F1 = 256

_CP_SC = pltpu.CompilerParams(needs_layout_passes=False)


@functools.cache
def _mesh():
    return plsc.VectorSubcoreMesh(core_axis_name="c", subcore_axis_name="s")
_HI = jax.lax.Precision.HIGHEST


def _wid():
    return lax.axis_index("s") * 2 + lax.axis_index("c")


def _dot(a, b, dims):
    return lax.dot_general(a, b, (dims, ((), ())),
                           preferred_element_type=jnp.float32, precision=_HI)


_GPT = 1280
_GCH = 128


def _sc_gather(tbl_hbm, idx_hbm, out_hbm, idx_v, rows_v, sem):
    w = _wid()
    base = w * _GPT
    pltpu.sync_copy(idx_hbm.at[pl.ds(base, _GPT)], idx_v)

    def body(c, carry):
        pltpu.async_copy(tbl_hbm.at[idx_v.at[pl.ds(c * _GCH, _GCH)]],
                         rows_v, sem).wait()
        pltpu.sync_copy(rows_v, out_hbm.at[pl.ds(base + c * _GCH, _GCH)])
        return carry

    lax.fori_loop(0, _GPT // _GCH, body, 0)


@jax.jit
def _gather_embeddings(tables, entity_ids):
    idx = (entity_ids.astype(jnp.int32)[None, :]
           + (jnp.arange(_K, dtype=jnp.int32) * _NE)[:, None]).reshape(-1)
    idx = jnp.pad(idx, (0, 32 * _GPT - _K * _N))
    k = functools.partial(
        pl.kernel, mesh=_mesh(), compiler_params=_CP_SC,
        out_type=jax.ShapeDtypeStruct((32 * _GPT, _D), jnp.float32),
        scratch_types=[
            pltpu.VMEM((_GPT,), jnp.int32),
            pltpu.VMEM((_GCH, _D), jnp.float32),
            pltpu.SemaphoreType.DMA,
        ],
    )(_sc_gather)
    rows = k(tables.reshape(_K * _NE, _D), idx)
    emb = rows[:_K * _N].reshape(_K, _N, _D)
    return jnp.pad(emb, ((0, 0), (0, _NP - _N), (0, 0)))


def _edge_loop(src_hbm, dst_hbm, si_v, di_v, es_v, ed_v, xlT_v, acc_v,
               den_v):
    def chunk(c, carry):
        pltpu.sync_copy(src_hbm.at[pl.ds(c * _CH, _CH)], si_v)
        pltpu.sync_copy(dst_hbm.at[pl.ds(c * _CH, _CH)], di_v)

        def vec(jb, carry2):
            base = jb * 80
            svs = [si_v[pl.ds(base + u * 16, 16)] for u in range(5)]
            dvs = [di_v[pl.ds(base + u * 16, 16)] for u in range(5)]
            wvs = []
            for u in range(5):
                e = (plsc.load_gather(es_v, [svs[u]])
                     + plsc.load_gather(ed_v, [dvs[u]]))
                wvs.append(jnp.exp(jnp.maximum(e, 0.2 * e)))
            for u in range(5):
                plsc.addupdate_scatter(den_v, [dvs[u]], wvs[u])
            for f in range(4):
                fo = f * _NP
                for u in range(5):
                    xv = plsc.load_gather(xlT_v, [svs[u] + fo])
                    plsc.addupdate_scatter(acc_v, [dvs[u] + fo],
                                           wvs[u] * xv)
            return carry2

        lax.fori_loop(0, (_CH // 16) // 5, vec, 0)
        return carry

    lax.fori_loop(0, _NCH, chunk, 0)


def _edge_unit(esd_hbm, xlT_hbm, src_hbm, dst_hbm, zeros_hbm, num_hbm,
               den_hbm, xlT_v, acc_v, es_v, ed_v, den_v, si_v, di_v,
               k, f0, h_es, h_ed, h_den, write_den):
    pltpu.sync_copy(esd_hbm.at[k, h_es], es_v)
    pltpu.sync_copy(esd_hbm.at[k, h_ed], ed_v)
    for r in range(4):
        pltpu.sync_copy(xlT_hbm.at[k, f0 + r], xlT_v.at[pl.ds(r * _NP, _NP)])
        pltpu.sync_copy(zeros_hbm, acc_v.at[pl.ds(r * _NP, _NP)])
    pltpu.sync_copy(zeros_hbm, den_v)
    _edge_loop(src_hbm, dst_hbm, si_v, di_v, es_v, ed_v, xlT_v, acc_v,
               den_v)
    for r in range(4):
        pltpu.sync_copy(acc_v.at[pl.ds(r * _NP, _NP)], num_hbm.at[k, f0 + r])

    @pl.when(write_den)
    def _():
        pltpu.sync_copy(den_v, den_hbm.at[k, h_den])


def _sc_edge1(xlT_hbm, esd_hbm, src_hbm, dst_hbm, zeros_hbm,
              num_hbm, den_hbm, xlT_v, acc_v, es_v, ed_v, den_v,
              si_v, di_v):
    w = _wid()
    for p in range(2):
        f0 = p * 128 + w * 4
        h = f0 // 32
        for k in range(_K):
            _edge_unit(esd_hbm, xlT_hbm, src_hbm, dst_hbm, zeros_hbm,
                       num_hbm, den_hbm, xlT_v, acc_v, es_v, ed_v, den_v,
                       si_v, di_v, k, f0, h, h + 8, h, f0 % 32 == 0)


def _sc_edge2(xlT_hbm, esd_hbm, src_hbm, dst_hbm, zeros_hbm,
              num_hbm, den_hbm, xlT_v, acc_v, es_v, ed_v, den_v,
              si_v, di_v):
    w = _wid()
    f0 = w * 4
    for k in range(_K):
        _edge_unit(esd_hbm, xlT_hbm, src_hbm, dst_hbm, zeros_hbm,
                   num_hbm, den_hbm, xlT_v, acc_v, es_v, ed_v, den_v,
                   si_v, di_v, k, f0, 0, 1, 0, f0 == 0)


def _edge_call(body, f_total, src, dst, xlT, esd, nheads):
    k = functools.partial(
        pl.kernel, mesh=_mesh(), compiler_params=_CP_SC,
        out_type=[
            jax.ShapeDtypeStruct((_K, f_total, _NP), jnp.float32),
            jax.ShapeDtypeStruct((_K, nheads, _NP), jnp.float32),
        ],
        scratch_types=[
            pltpu.VMEM((4 * _NP,), jnp.float32),
            pltpu.VMEM((4 * _NP,), jnp.float32),
            pltpu.VMEM((_NP,), jnp.float32),
            pltpu.VMEM((_NP,), jnp.float32),
            pltpu.VMEM((_NP,), jnp.float32),
            pltpu.VMEM((_CH,), jnp.int32),
            pltpu.VMEM((_CH,), jnp.int32),
        ],
    )(body)
    zeros = jnp.zeros((_NP,), jnp.float32)
    return k(xlT, esd, src, dst, zeros)


def _ln_t(x, g, b):
    m = jnp.mean(x, axis=0, keepdims=True)
    v = jnp.mean((x - m) ** 2, axis=0, keepdims=True)
    return (x - m) / jnp.sqrt(v + 1e-5) * g + b


def _elu(x):
    return jnp.where(x > 0, x, jnp.exp(jnp.minimum(x, 0.0)) - 1.0)


def _k2_body(emb_ref, w1_ref, a1_ref, xlT_ref, esd_ref):
    e = emb_ref[0]
    xlT = _dot(w1_ref[0], e, ((0,), (1,)))
    xlT_ref[0] = xlT
    esd_ref[0] = _dot(a1_ref[0], xlT, ((0,), (0,)))


@jax.jit
def _tc_k2(emb, W1, A1):
    return pl.pallas_call(
        _k2_body,
        grid=(_K, _GN),
        in_specs=[
            pl.BlockSpec((1, _NB, _D), lambda k, n: (k, n, 0)),
            pl.BlockSpec((1, _D, _F1), lambda k, n: (k, 0, 0)),
            pl.BlockSpec((1, _F1, 16), lambda k, n: (k, 0, 0)),
        ],
        out_specs=[
            pl.BlockSpec((1, _F1, _NB), lambda k, n: (k, 0, n)),
            pl.BlockSpec((1, 16, _NB), lambda k, n: (k, 0, n)),
        ],
        out_shape=[
            jax.ShapeDtypeStruct((_K, _F1, _NP), jnp.float32),
            jax.ShapeDtypeStruct((_K, 16, _NP), jnp.float32),
        ],
    )(emb, W1, A1)


def _k4_body(num_ref, den_ref, e8_ref, b1_ref, g1_ref, bb1_ref, w2_ref,
             a2_ref, xl2_ref, esd2_ref):
    dene = _dot(e8_ref[...], den_ref[0], ((1,), (0,)))
    o = num_ref[0] / (dene + 1e-16) + b1_ref[0]
    h = _elu(_ln_t(o, g1_ref[0], bb1_ref[0]))
    xl2 = _dot(w2_ref[0], h, ((0,), (0,)))
    xl2_ref[0] = xl2
    esd2_ref[0] = _dot(a2_ref[0], xl2, ((0,), (0,)))


@jax.jit
def _tc_k4(num1, den1, E8, b1c, ln1gc, ln1bc, W2, A2):
    return pl.pallas_call(
        _k4_body,
        grid=(_K, _GN),
        in_specs=[
            pl.BlockSpec((1, _F1, _NB), lambda k, n: (k, 0, n)),
            pl.BlockSpec((1, 8, _NB), lambda k, n: (k, 0, n)),
            pl.BlockSpec((_F1, 8), lambda k, n: (0, 0)),
            pl.BlockSpec((1, _F1, 1), lambda k, n: (k, 0, 0)),
            pl.BlockSpec((1, _F1, 1), lambda k, n: (k, 0, 0)),
            pl.BlockSpec((1, _F1, 1), lambda k, n: (k, 0, 0)),
            pl.BlockSpec((1, _F1, _D), lambda k, n: (k, 0, 0)),
            pl.BlockSpec((1, _D, 16), lambda k, n: (k, 0, 0)),
        ],
        out_specs=[
            pl.BlockSpec((1, _D, _NB), lambda k, n: (k, 0, n)),
            pl.BlockSpec((1, 16, _NB), lambda k, n: (k, 0, n)),
        ],
        out_shape=[
            jax.ShapeDtypeStruct((_K, _D, _NP), jnp.float32),
            jax.ShapeDtypeStruct((_K, 16, _NP), jnp.float32),
        ],
    )(num1, den1, E8, b1c, ln1gc, ln1bc, W2, A2)


def _k6_body(num_ref, den_ref, b2_ref, g2_ref, bb2_ref, pw1_ref, pb1_ref,
             pg_ref, pbb_ref, pw2_ref, pb2_ref, kg_ref, mw_ref, mb_ref,
             gx_ref, gc_ref, gb_ref, s8_ref, fw1_ref, fb1_ref, fg_ref,
             fbb_ref, fw2_ref, fb2_ref, i128_ref, out_ref):
    s8 = s8_ref[...]
    xs = []
    for k in range(_K):
        o2 = num_ref[k] / (den_ref[k] + 1e-16) + b2_ref[k]
        h2 = _elu(_ln_t(o2, g2_ref[k], bb2_ref[k]))
        t = _dot(pw1_ref[k], h2, ((0,), (0,))) + pb1_ref[k]
        t = jnp.maximum(_ln_t(t, pg_ref[k], pbb_ref[k]), 0.0)
        p = _dot(pw2_ref[k], t, ((0,), (0,))) + pb2_ref[k]
        xs.append(p + kg_ref[k])

    def mha(seq, layer):
        qs = [_dot(mw_ref[layer, 0], x, ((0,), (0,))) + mb_ref[layer, 0]
              for x in seq]
        ks = [_dot(mw_ref[layer, 1], x, ((0,), (0,))) + mb_ref[layer, 1]
              for x in seq]
        vs = [_dot(mw_ref[layer, 2], x, ((0,), (0,))) + mb_ref[layer, 2]
              for x in seq]
        outs = []
        for i in range(4):
            sc = [_dot(s8, qs[i] * ks[j], ((0,), (0,))) * 0.25
                  for j in range(4)]
            m = jnp.maximum(jnp.maximum(sc[0], sc[1]),
                            jnp.maximum(sc[2], sc[3]))
            ex = [jnp.exp(s - m) for s in sc]
            tot = ex[0] + ex[1] + ex[2] + ex[3]
            o = sum(_dot(s8, ex[j] / tot, ((1,), (0,))) * vs[j]
                    for j in range(4))
            outs.append(_dot(mw_ref[layer, 3], o, ((0,), (0,)))
                        + mb_ref[layer, 3])
        return outs

    intra = mha(xs, 0)
    cross = mha(intra, 1)
    fused = []
    for s in range(4):
        g = jax.nn.sigmoid(_dot(gx_ref[...], xs[s], ((0,), (0,)))
                           + _dot(gc_ref[...], cross[s], ((0,), (0,)))
                           + gb_ref[...])
        fused.append(g * cross[s] + (1.0 - g) * xs[s])
    fcat = jnp.concatenate(fused, axis=0)
    t = _dot(fw1_ref[...], fcat, ((0,), (0,))) + fb1_ref[...]
    t = jnp.maximum(_ln_t(t, fg_ref[...], fbb_ref[...]), 0.0)
    oT = _dot(fw2_ref[...], t, ((0,), (0,))) + fb2_ref[...]
    out_ref[...] = _dot(oT, i128_ref[...], ((0,), (0,)))


@jax.jit
def _tc_k6(num2, den2, b2c, ln2gc, ln2bc, pw1, pb1c, plngc, plnbc, pw2,
           pb2c, kgc, mha_w, mha_bc, gx, gc, gbc, S8, fw1, fb1c, flngc,
           flnbc, fw2, fb2c, I128):
    full = lambda *shape: pl.BlockSpec(shape, lambda n: (0,) * len(shape))
    return pl.pallas_call(
        _k6_body,
        grid=(_GN,),
        in_specs=[
            pl.BlockSpec((_K, _D, _NB), lambda n: (0, 0, n)),
            pl.BlockSpec((_K, 1, _NB), lambda n: (0, 0, n)),
            full(_K, _D, 1), full(_K, _D, 1), full(_K, _D, 1),
            full(_K, _D, 32), full(_K, 32, 1), full(_K, 32, 1),
            full(_K, 32, 1), full(_K, 32, _D), full(_K, _D, 1),
            full(_K, _D, 1),
            full(2, 4, _D, _D), full(2, 4, _D, 1),
            full(_D, _D), full(_D, _D), full(_D, 1),
            full(_D, 8),
            full(4 * _D, 32), full(32, 1), full(32, 1), full(32, 1),
            full(32, _D), full(_D, 1),
            full(_D, _D),
        ],
        out_specs=pl.BlockSpec((_NB, _D), lambda n: (n, 0)),
        out_shape=jax.ShapeDtypeStruct((_NP, _D), jnp.float32),
    )(num2, den2, b2c, ln2gc, ln2bc, pw1, pb1c, plngc, plnbc, pw2, pb2c,
      kgc, mha_w, mha_bc, gx, gc, gbc, S8, fw1, fb1c, flngc, flnbc, fw2,
      fb2c, I128)


def kernel(entity_ids, edge_index, tables, W1, b1, a1s, a1d, ln1g, ln1b,
           W2, b2, a2s, a2d, ln2g, ln2b, pw1, pb1, plng, plnb, pw2, pb2,
           kg_emb, mha_w, mha_b, gate_w, gate_b, fw1, fb1, flng, flnb,
           fw2, fb2):
    f32 = jnp.float32
    src = jnp.concatenate([edge_index[0].astype(jnp.int32),
                           jnp.arange(_N, dtype=jnp.int32)])
    dst = jnp.concatenate([edge_index[1].astype(jnp.int32),
                           jnp.arange(_N, dtype=jnp.int32)])

    eye8 = jnp.eye(8, dtype=f32)
    A1 = jnp.concatenate([
        (a1s[:, :, :, None] * eye8[:, None, :]).reshape(_K, _F1, 8),
        (a1d[:, :, :, None] * eye8[:, None, :]).reshape(_K, _F1, 8),
    ], axis=2)
    A2 = jnp.zeros((_K, _D, 16), f32)
    A2 = A2.at[:, :, 0].set(a2s[:, 0, :]).at[:, :, 1].set(a2d[:, 0, :])
    E8 = jnp.kron(jnp.eye(8, dtype=f32), jnp.ones((32, 1), f32))
    S8 = jnp.kron(jnp.eye(8, dtype=f32), jnp.ones((16, 1), f32))
    I128 = jnp.eye(_D, dtype=f32)

    col = lambda a: a[..., None]

    emb = _gather_embeddings(tables, entity_ids)

    xl1T, esd1 = _tc_k2(emb, W1, A1)
    num1, den1 = _edge_call(_sc_edge1, _F1, src, dst, xl1T, esd1, 8)
    xl2T, esd2 = _tc_k4(num1, den1, E8, col(b1), col(ln1g), col(ln1b),
                        W2, A2)
    num2, den2 = _edge_call(_sc_edge2, _D, src, dst, xl2T, esd2, 1)

    o = _tc_k6(num2, den2, col(b2), col(ln2g), col(ln2b),
               pw1, col(pb1), col(plng), col(plnb), pw2, col(pb2),
               col(kg_emb), mha_w, col(mha_b), gate_w[:_D], gate_w[_D:],
               col(gate_b), S8, fw1, col(fb1), col(flng), col(flnb),
               fw2, col(fb2), I128)
    return o[:_N]


# speedup vs baseline: 1.3034x; 1.0114x over previous
import functools
import jax
import jax.numpy as jnp
from jax import lax
from jax.experimental import pallas as pl
from jax.experimental.pallas import tpu as pltpu
from jax.experimental.pallas import tpu_sc as plsc

_N = 10000
_D = 128
_K = 4
_NE = 100000
_NP = 10240
_EP = 170000
_NB = 128
_GN = _NP // _NB
_CH = 6800
_NCH = _EP // _CH
_F1 = 256

_CP_SC = pltpu.CompilerParams(needs_layout_passes=False)


@functools.cache
def _mesh():
    return plsc.VectorSubcoreMesh(core_axis_name="c", subcore_axis_name="s")
_HI = jax.lax.Precision.HIGHEST


def _wid():
    return lax.axis_index("s") * 2 + lax.axis_index("c")


def _dot(a, b, dims):
    return lax.dot_general(a, b, (dims, ((), ())),
                           preferred_element_type=jnp.float32, precision=_HI)


_GPT = 1280
_GCH = 128


def _sc_gather(tbl_hbm, idx_hbm, out_hbm, idx_v, rows_v, sem):
    w = _wid()
    base = w * _GPT
    pltpu.sync_copy(idx_hbm.at[pl.ds(base, _GPT)], idx_v)

    def body(c, carry):
        pltpu.async_copy(tbl_hbm.at[idx_v.at[pl.ds(c * _GCH, _GCH)]],
                         rows_v, sem).wait()
        pltpu.sync_copy(rows_v, out_hbm.at[pl.ds(base + c * _GCH, _GCH)])
        return carry

    lax.fori_loop(0, _GPT // _GCH, body, 0)


@jax.jit
def _gather_embeddings(tables, entity_ids):
    idx = (entity_ids.astype(jnp.int32)[None, :]
           + (jnp.arange(_K, dtype=jnp.int32) * _NE)[:, None]).reshape(-1)
    idx = jnp.pad(idx, (0, 32 * _GPT - _K * _N))
    k = functools.partial(
        pl.kernel, mesh=_mesh(), compiler_params=_CP_SC,
        out_type=jax.ShapeDtypeStruct((32 * _GPT, _D), jnp.float32),
        scratch_types=[
            pltpu.VMEM((_GPT,), jnp.int32),
            pltpu.VMEM((_GCH, _D), jnp.float32),
            pltpu.SemaphoreType.DMA,
        ],
    )(_sc_gather)
    rows = k(tables.reshape(_K * _NE, _D), idx)
    emb = rows[:_K * _N].reshape(_K, _N, _D)
    return jnp.pad(emb, ((0, 0), (0, _NP - _N), (0, 0)))


def _edge_loop(src_hbm, dst_hbm, wT_hbm, si_v, di_v, wb_v, xlT_v, acc_v,
               k, h):
    def chunk(c, carry):
        pltpu.sync_copy(src_hbm.at[pl.ds(c * _CH, _CH)], si_v)
        pltpu.sync_copy(dst_hbm.at[pl.ds(c * _CH, _CH)], di_v)
        pltpu.sync_copy(wT_hbm.at[k, h, c], wb_v)

        def vec(jb, carry2):
            base = jb * 80
            svs = [si_v[pl.ds(base + u * 16, 16)] for u in range(5)]
            dvs = [di_v[pl.ds(base + u * 16, 16)] for u in range(5)]
            wvs = [wb_v[pl.ds(base + u * 16, 16)] for u in range(5)]
            for f in range(4):
                fo = f * _NP
                for u in range(5):
                    xv = plsc.load_gather(xlT_v, [svs[u] + fo])
                    plsc.addupdate_scatter(acc_v, [dvs[u] + fo],
                                           wvs[u] * xv)
            return carry2

        lax.fori_loop(0, (_CH // 16) // 5, vec, 0)
        return carry

    lax.fori_loop(0, _NCH, chunk, 0)


def _w_scan(si_v, di_v, es_v, ed_v, wb_v, den_v):
    def vec(jb, carry2):
        base = jb * 80
        svs = [si_v[pl.ds(base + u * 16, 16)] for u in range(5)]
        dvs = [di_v[pl.ds(base + u * 16, 16)] for u in range(5)]
        for u in range(5):
            e = (plsc.load_gather(es_v, [svs[u]])
                 + plsc.load_gather(ed_v, [dvs[u]]))
            wv = jnp.exp(jnp.maximum(e, 0.2 * e))
            wb_v[pl.ds(base + u * 16, 16)] = wv
            plsc.addupdate_scatter(den_v, [dvs[u]], wv)
        return carry2

    lax.fori_loop(0, (_CH // 16) // 5, vec, 0)


def _sc_w1(esd_hbm, src_hbm, dst_hbm, zeros_hbm, wT_hbm, denp_hbm,
           es_v, ed_v, den_v, si_v, di_v, wb_v):
    w = _wid()
    h = w // 4
    q = w % 4
    for k in range(_K):
        pltpu.sync_copy(esd_hbm.at[k, h], es_v)
        pltpu.sync_copy(esd_hbm.at[k, h + 8], ed_v)
        pltpu.sync_copy(zeros_hbm, den_v)

        def chunk(i, carry):
            c = q + 4 * i

            @pl.when(c < _NCH)
            def _():
                pltpu.sync_copy(src_hbm.at[pl.ds(c * _CH, _CH)], si_v)
                pltpu.sync_copy(dst_hbm.at[pl.ds(c * _CH, _CH)], di_v)
                _w_scan(si_v, di_v, es_v, ed_v, wb_v, den_v)
                pltpu.sync_copy(wb_v, wT_hbm.at[k, h, c])
            return carry

        lax.fori_loop(0, 7, chunk, 0)
        pltpu.sync_copy(den_v, denp_hbm.at[k, q, h])


def _sc_w2(esd_hbm, src_hbm, dst_hbm, zeros_hbm, wT_hbm, denp_hbm,
           es_v, ed_v, den_v, si_v, di_v, wb_v):
    w = _wid()

    @pl.when(w < _NCH)
    def _():
        for k in range(_K):
            pltpu.sync_copy(esd_hbm.at[k, 0], es_v)
            pltpu.sync_copy(esd_hbm.at[k, 1], ed_v)
            pltpu.sync_copy(zeros_hbm, den_v)
            pltpu.sync_copy(src_hbm.at[pl.ds(w * _CH, _CH)], si_v)
            pltpu.sync_copy(dst_hbm.at[pl.ds(w * _CH, _CH)], di_v)
            _w_scan(si_v, di_v, es_v, ed_v, wb_v, den_v)
            pltpu.sync_copy(wb_v, wT_hbm.at[k, 0, w])
            pltpu.sync_copy(den_v, denp_hbm.at[k, w])


def _w_call(body, src, dst, esd, nheads, nparts):
    k = functools.partial(
        pl.kernel, mesh=_mesh(), compiler_params=_CP_SC,
        out_type=[
            jax.ShapeDtypeStruct((_K, nheads, _NCH, _CH), jnp.float32),
            jax.ShapeDtypeStruct((_K, nparts, nheads, _NP), jnp.float32)
            if nheads > 1 else
            jax.ShapeDtypeStruct((_K, nparts, _NP), jnp.float32),
        ],
        scratch_types=[
            pltpu.VMEM((_NP,), jnp.float32),
            pltpu.VMEM((_NP,), jnp.float32),
            pltpu.VMEM((_NP,), jnp.float32),
            pltpu.VMEM((_CH,), jnp.int32),
            pltpu.VMEM((_CH,), jnp.int32),
            pltpu.VMEM((_CH,), jnp.float32),
        ],
    )(body)
    zeros = jnp.zeros((_NP,), jnp.float32)
    return k(esd, src, dst, zeros)


def _edge_unit(xlT_hbm, src_hbm, dst_hbm, wT_hbm, zeros_hbm, num_hbm,
               xlT_v, acc_v, si_v, di_v, wb_v, k, f0, h):
    for r in range(4):
        pltpu.sync_copy(xlT_hbm.at[k, f0 + r], xlT_v.at[pl.ds(r * _NP, _NP)])
        pltpu.sync_copy(zeros_hbm, acc_v.at[pl.ds(r * _NP, _NP)])
    _edge_loop(src_hbm, dst_hbm, wT_hbm, si_v, di_v, wb_v, xlT_v, acc_v,
               k, h)
    for r in range(4):
        pltpu.sync_copy(acc_v.at[pl.ds(r * _NP, _NP)], num_hbm.at[k, f0 + r])


def _sc_edge1(xlT_hbm, src_hbm, dst_hbm, wT_hbm, zeros_hbm, num_hbm,
              xlT_v, acc_v, si_v, di_v, wb_v):
    w = _wid()
    for p in range(2):
        f0 = p * 128 + w * 4
        h = f0 // 32
        for k in range(_K):
            _edge_unit(xlT_hbm, src_hbm, dst_hbm, wT_hbm, zeros_hbm,
                       num_hbm, xlT_v, acc_v, si_v, di_v, wb_v, k, f0, h)


def _sc_edge2(xlT_hbm, src_hbm, dst_hbm, wT_hbm, zeros_hbm, num_hbm,
              xlT_v, acc_v, si_v, di_v, wb_v):
    w = _wid()
    f0 = w * 4
    for k in range(_K):
        _edge_unit(xlT_hbm, src_hbm, dst_hbm, wT_hbm, zeros_hbm, num_hbm,
                   xlT_v, acc_v, si_v, di_v, wb_v, k, f0, 0)


def _edge_call(body, f_total, src, dst, xlT, wT):
    k = functools.partial(
        pl.kernel, mesh=_mesh(), compiler_params=_CP_SC,
        out_type=jax.ShapeDtypeStruct((_K, f_total, _NP), jnp.float32),
        scratch_types=[
            pltpu.VMEM((4 * _NP,), jnp.float32),
            pltpu.VMEM((4 * _NP,), jnp.float32),
            pltpu.VMEM((_CH,), jnp.int32),
            pltpu.VMEM((_CH,), jnp.int32),
            pltpu.VMEM((_CH,), jnp.float32),
        ],
    )(body)
    zeros = jnp.zeros((_NP,), jnp.float32)
    return k(xlT, src, dst, wT, zeros)


def _ln_t(x, g, b):
    m = jnp.mean(x, axis=0, keepdims=True)
    v = jnp.mean((x - m) ** 2, axis=0, keepdims=True)
    return (x - m) / jnp.sqrt(v + 1e-5) * g + b


def _elu(x):
    return jnp.where(x > 0, x, jnp.exp(jnp.minimum(x, 0.0)) - 1.0)


def _k2_body(emb_ref, w1_ref, a1_ref, xlT_ref, esd_ref):
    e = emb_ref[0]
    xlT = _dot(w1_ref[0], e, ((0,), (1,)))
    xlT_ref[0] = xlT
    esd_ref[0] = _dot(a1_ref[0], xlT, ((0,), (0,)))


@jax.jit
def _tc_k2(emb, W1, A1):
    return pl.pallas_call(
        _k2_body,
        grid=(_K, _GN),
        in_specs=[
            pl.BlockSpec((1, _NB, _D), lambda k, n: (k, n, 0)),
            pl.BlockSpec((1, _D, _F1), lambda k, n: (k, 0, 0)),
            pl.BlockSpec((1, _F1, 16), lambda k, n: (k, 0, 0)),
        ],
        out_specs=[
            pl.BlockSpec((1, _F1, _NB), lambda k, n: (k, 0, n)),
            pl.BlockSpec((1, 16, _NB), lambda k, n: (k, 0, n)),
        ],
        out_shape=[
            jax.ShapeDtypeStruct((_K, _F1, _NP), jnp.float32),
            jax.ShapeDtypeStruct((_K, 16, _NP), jnp.float32),
        ],
    )(emb, W1, A1)


def _k4_body(num_ref, den_ref, e8_ref, b1_ref, g1_ref, bb1_ref, w2_ref,
             a2_ref, xl2_ref, esd2_ref):
    den = jnp.sum(den_ref[0], axis=0)
    dene = _dot(e8_ref[...], den, ((1,), (0,)))
    o = num_ref[0] / (dene + 1e-16) + b1_ref[0]
    h = _elu(_ln_t(o, g1_ref[0], bb1_ref[0]))
    xl2 = _dot(w2_ref[0], h, ((0,), (0,)))
    xl2_ref[0] = xl2
    esd2_ref[0] = _dot(a2_ref[0], xl2, ((0,), (0,)))


@jax.jit
def _tc_k4(num1, den1, E8, b1c, ln1gc, ln1bc, W2, A2):
    return pl.pallas_call(
        _k4_body,
        grid=(_K, _GN),
        in_specs=[
            pl.BlockSpec((1, _F1, _NB), lambda k, n: (k, 0, n)),
            pl.BlockSpec((1, 4, 8, _NB), lambda k, n: (k, 0, 0, n)),
            pl.BlockSpec((_F1, 8), lambda k, n: (0, 0)),
            pl.BlockSpec((1, _F1, 1), lambda k, n: (k, 0, 0)),
            pl.BlockSpec((1, _F1, 1), lambda k, n: (k, 0, 0)),
            pl.BlockSpec((1, _F1, 1), lambda k, n: (k, 0, 0)),
            pl.BlockSpec((1, _F1, _D), lambda k, n: (k, 0, 0)),
            pl.BlockSpec((1, _D, 16), lambda k, n: (k, 0, 0)),
        ],
        out_specs=[
            pl.BlockSpec((1, _D, _NB), lambda k, n: (k, 0, n)),
            pl.BlockSpec((1, 16, _NB), lambda k, n: (k, 0, n)),
        ],
        out_shape=[
            jax.ShapeDtypeStruct((_K, _D, _NP), jnp.float32),
            jax.ShapeDtypeStruct((_K, 16, _NP), jnp.float32),
        ],
    )(num1, den1, E8, b1c, ln1gc, ln1bc, W2, A2)


def _k6_body(num_ref, den_ref, b2_ref, g2_ref, bb2_ref, pw1_ref, pb1_ref,
             pg_ref, pbb_ref, pw2_ref, pb2_ref, kg_ref, mw_ref, mb_ref,
             gx_ref, gc_ref, gb_ref, s8_ref, fw1_ref, fb1_ref, fg_ref,
             fbb_ref, fw2_ref, fb2_ref, i128_ref, out_ref):
    s8 = s8_ref[...]
    xs = []
    for k in range(_K):
        den = jnp.sum(den_ref[k], axis=0, keepdims=True)
        o2 = num_ref[k] / (den + 1e-16) + b2_ref[k]
        h2 = _elu(_ln_t(o2, g2_ref[k], bb2_ref[k]))
        t = _dot(pw1_ref[k], h2, ((0,), (0,))) + pb1_ref[k]
        t = jnp.maximum(_ln_t(t, pg_ref[k], pbb_ref[k]), 0.0)
        p = _dot(pw2_ref[k], t, ((0,), (0,))) + pb2_ref[k]
        xs.append(p + kg_ref[k])

    def mha(seq, layer):
        qs = [_dot(mw_ref[layer, 0], x, ((0,), (0,))) + mb_ref[layer, 0]
              for x in seq]
        ks = [_dot(mw_ref[layer, 1], x, ((0,), (0,))) + mb_ref[layer, 1]
              for x in seq]
        vs = [_dot(mw_ref[layer, 2], x, ((0,), (0,))) + mb_ref[layer, 2]
              for x in seq]
        outs = []
        for i in range(4):
            sc = [_dot(s8, qs[i] * ks[j], ((0,), (0,))) * 0.25
                  for j in range(4)]
            m = jnp.maximum(jnp.maximum(sc[0], sc[1]),
                            jnp.maximum(sc[2], sc[3]))
            ex = [jnp.exp(s - m) for s in sc]
            tot = ex[0] + ex[1] + ex[2] + ex[3]
            o = sum(_dot(s8, ex[j] / tot, ((1,), (0,))) * vs[j]
                    for j in range(4))
            outs.append(_dot(mw_ref[layer, 3], o, ((0,), (0,)))
                        + mb_ref[layer, 3])
        return outs

    intra = mha(xs, 0)
    cross = mha(intra, 1)
    fused = []
    for s in range(4):
        g = jax.nn.sigmoid(_dot(gx_ref[...], xs[s], ((0,), (0,)))
                           + _dot(gc_ref[...], cross[s], ((0,), (0,)))
                           + gb_ref[...])
        fused.append(g * cross[s] + (1.0 - g) * xs[s])
    fcat = jnp.concatenate(fused, axis=0)
    t = _dot(fw1_ref[...], fcat, ((0,), (0,))) + fb1_ref[...]
    t = jnp.maximum(_ln_t(t, fg_ref[...], fbb_ref[...]), 0.0)
    oT = _dot(fw2_ref[...], t, ((0,), (0,))) + fb2_ref[...]
    out_ref[...] = _dot(oT, i128_ref[...], ((0,), (0,)))


@jax.jit
def _tc_k6(num2, den2, b2c, ln2gc, ln2bc, pw1, pb1c, plngc, plnbc, pw2,
           pb2c, kgc, mha_w, mha_bc, gx, gc, gbc, S8, fw1, fb1c, flngc,
           flnbc, fw2, fb2c, I128):
    full = lambda *shape: pl.BlockSpec(shape, lambda n: (0,) * len(shape))
    return pl.pallas_call(
        _k6_body,
        grid=(_GN,),
        in_specs=[
            pl.BlockSpec((_K, _D, _NB), lambda n: (0, 0, n)),
            pl.BlockSpec((_K, 25, _NB), lambda n: (0, 0, n)),
            full(_K, _D, 1), full(_K, _D, 1), full(_K, _D, 1),
            full(_K, _D, 32), full(_K, 32, 1), full(_K, 32, 1),
            full(_K, 32, 1), full(_K, 32, _D), full(_K, _D, 1),
            full(_K, _D, 1),
            full(2, 4, _D, _D), full(2, 4, _D, 1),
            full(_D, _D), full(_D, _D), full(_D, 1),
            full(_D, 8),
            full(4 * _D, 32), full(32, 1), full(32, 1), full(32, 1),
            full(32, _D), full(_D, 1),
            full(_D, _D),
        ],
        out_specs=pl.BlockSpec((_NB, _D), lambda n: (n, 0)),
        out_shape=jax.ShapeDtypeStruct((_NP, _D), jnp.float32),
    )(num2, den2, b2c, ln2gc, ln2bc, pw1, pb1c, plngc, plnbc, pw2, pb2c,
      kgc, mha_w, mha_bc, gx, gc, gbc, S8, fw1, fb1c, flngc, flnbc, fw2,
      fb2c, I128)


def kernel(entity_ids, edge_index, tables, W1, b1, a1s, a1d, ln1g, ln1b,
           W2, b2, a2s, a2d, ln2g, ln2b, pw1, pb1, plng, plnb, pw2, pb2,
           kg_emb, mha_w, mha_b, gate_w, gate_b, fw1, fb1, flng, flnb,
           fw2, fb2):
    f32 = jnp.float32
    src = jnp.concatenate([edge_index[0].astype(jnp.int32),
                           jnp.arange(_N, dtype=jnp.int32)])
    dst = jnp.concatenate([edge_index[1].astype(jnp.int32),
                           jnp.arange(_N, dtype=jnp.int32)])

    eye8 = jnp.eye(8, dtype=f32)
    A1 = jnp.concatenate([
        (a1s[:, :, :, None] * eye8[:, None, :]).reshape(_K, _F1, 8),
        (a1d[:, :, :, None] * eye8[:, None, :]).reshape(_K, _F1, 8),
    ], axis=2)
    A2 = jnp.zeros((_K, _D, 16), f32)
    A2 = A2.at[:, :, 0].set(a2s[:, 0, :]).at[:, :, 1].set(a2d[:, 0, :])
    E8 = jnp.kron(jnp.eye(8, dtype=f32), jnp.ones((32, 1), f32))
    S8 = jnp.kron(jnp.eye(8, dtype=f32), jnp.ones((16, 1), f32))
    I128 = jnp.eye(_D, dtype=f32)

    col = lambda a: a[..., None]

    emb = _gather_embeddings(tables, entity_ids)

    xl1T, esd1 = _tc_k2(emb, W1, A1)
    wT1, den1 = _w_call(_sc_w1, src, dst, esd1, 8, 4)
    num1 = _edge_call(_sc_edge1, _F1, src, dst, xl1T, wT1)
    xl2T, esd2 = _tc_k4(num1, den1, E8, col(b1), col(ln1g), col(ln1b),
                        W2, A2)
    wT2, den2 = _w_call(_sc_w2, src, dst, esd2, 1, _NCH)
    num2 = _edge_call(_sc_edge2, _D, src, dst, xl2T, wT2)

    o = _tc_k6(num2, den2, col(b2), col(ln2g), col(ln2b),
               pw1, col(pb1), col(plng), col(plnb), pw2, col(pb2),
               col(kg_emb), mha_w, col(mha_b), gate_w[:_D], gate_w[_D:],
               col(gate_b), S8, fw1, col(fb1), col(flng), col(flnb),
               fw2, col(fb2), I128)
    return o[:_N]
